# Initial kernel scaffold; baseline (speedup 1.0000x reference)
import functools

import jax
import jax.numpy as jnp
from jax import lax
from jax.experimental import pallas as pl
from jax.experimental.pallas import tpu as pltpu
from jax.experimental.pallas import tpu_sc as plsc

N = 10000
E = 320000
F = 128
NW = 32
RPW = 79
ROWS_PAD = NW * RPW
EPAD = ROWS_PAD * 128
NPAD = 10240


def _leaky2(x):
    return jnp.maximum(x, 0.2 * x)


def _pre_body(hv, p, q, he, Wa, ba, We1, We2, We3, We4, be, wp, wq, whe, bal,
              lr_o, r_o, s_o, t_o, hw_o):
    hvb, pb, qb = hv[...], p[...], q[...]
    p2 = jnp.dot(pb, We2[...])
    q3 = jnp.dot(qb, We3[...])
    lr_o[...] = _leaky2(jnp.dot(hvb, Wa[...]) + ba[...])
    r_o[...] = jnp.dot(hvb, We1[...]) - p2 - q3 + be[...]
    s_o[...] = p2 + q3 + jnp.dot(hvb, We4[...])
    t_o[...] = jnp.dot(pb, wp[...]) + jnp.dot(qb, wq[...])
    hw_o[...] = jnp.dot(he[...], whe[...]) + bal[...]


def _tc_precompute(hv, p, q, he, Wa, ba, We1, We2, We3, We4, be, wp, wq, whe, bal):
    nb = 1000
    eb = 32000
    grid = (N // nb,)
    node_in = pl.BlockSpec((nb, F), lambda i: (i, 0))
    full = lambda shape: pl.BlockSpec(shape, lambda i: tuple(0 for _ in shape))
    return pl.pallas_call(
        _pre_body,
        grid=grid,
        in_specs=[
            node_in, node_in, node_in,
            pl.BlockSpec((eb, 16), lambda i: (i, 0)),
            full((F, F)), full((F,)),
            full((F, F)), full((F, F)), full((F, F)), full((F, F)), full((F,)),
            full((F, 1)), full((F, 1)), full((16, 1)), full((1,)),
        ],
        out_specs=[
            pl.BlockSpec((nb, F), lambda i: (i, 0)),
            pl.BlockSpec((nb, F), lambda i: (i, 0)),
            pl.BlockSpec((nb, F), lambda i: (i, 0)),
            pl.BlockSpec((nb, 1), lambda i: (i, 0)),
            pl.BlockSpec((eb, 1), lambda i: (i, 0)),
        ],
        out_shape=[
            jax.ShapeDtypeStruct((N, F), jnp.float32),
            jax.ShapeDtypeStruct((N, F), jnp.float32),
            jax.ShapeDtypeStruct((N, F), jnp.float32),
            jax.ShapeDtypeStruct((N, 1), jnp.float32),
            jax.ShapeDtypeStruct((E, 1), jnp.float32),
        ],
    )(hv, p, q, he, Wa, ba, We1, We2, We3, We4, be, wp, wq, whe, bal)


def _sc_body(recv2, send2, hw2, t_hbm, lr_hbm, r_hbm, s_hbm,
             me_o, mv_o, den_o,
             t_v, ridx, sidx, hw_v, rows_l, rows_a, rows_b, exrow, zden,
             mv_acc, den_acc, sem1, sem2, sem3):
    c = lax.axis_index("c")
    s = lax.axis_index("s")
    wid = c * 16 + s
    base = wid * RPW

    zv = jnp.zeros((16,), jnp.float32)

    def _zrow(e, _):
        for v in range(8):
            rows_l[e, pl.ds(v * 16, 16)] = zv
        return 0

    lax.fori_loop(0, 128, _zrow, 0)

    def _zden(i, _):
        zden[pl.ds(i * 16, 16)] = zv
        return 0

    lax.fori_loop(0, 40, _zden, 0)

    for k in range(5):
        pltpu.sync_copy(rows_l.at[pl.ds(0, 125)],
                        mv_acc.at[pl.ds(s * 625 + k * 125, 125)])
    pltpu.sync_copy(zden, den_acc.at[pl.ds(s * 640, 640)])
    plsc.subcore_barrier()

    pltpu.sync_copy(t_hbm, t_v)
    pltpu.sync_copy(recv2.at[pl.ds(base, RPW)], ridx)
    pltpu.sync_copy(send2.at[pl.ds(base, RPW)], sidx)
    pltpu.sync_copy(hw2.at[pl.ds(base, RPW)], hw_v)

    def _row(r, _):
        si_ref = sidx.at[r]
        ri_ref = ridx.at[r]
        c_l = pltpu.async_copy(lr_hbm.at[si_ref], rows_l, sem1)
        c_a = pltpu.async_copy(r_hbm.at[ri_ref], rows_a, sem2)
        c_b = pltpu.async_copy(s_hbm.at[si_ref], rows_b, sem3)

        def _ex(g, _):
            si = sidx[r, pl.ds(g * 16, 16)]
            ri = ridx[r, pl.ds(g * 16, 16)]
            ts = plsc.load_gather(t_v, [si])
            tr = plsc.load_gather(t_v, [ri])
            hwv = hw_v[r, pl.ds(g * 16, 16)]
            exrow[0, pl.ds(g * 16, 16)] = jnp.exp(ts - tr + hwv)
            return 0

        lax.fori_loop(0, 8, _ex, 0)
        pltpu.sync_copy(exrow.at[0], den_acc.at[ri_ref], add=True)

        c_l.wait()
        zi = jnp.zeros((16,), jnp.int32)

        def _scale(e, _):
            exb = plsc.load_gather(exrow, [zi, jnp.full((16,), e, jnp.int32)])
            for v in range(8):
                rows_l[e, pl.ds(v * 16, 16)] = rows_l[e, pl.ds(v * 16, 16)] * exb
            return 0

        lax.fori_loop(0, 128, _scale, 0)
        pltpu.sync_copy(rows_l, mv_acc.at[ri_ref], add=True)

        c_a.wait()
        c_b.wait()

        def _me(e, _):
            for v in range(8):
                a = rows_a[e, pl.ds(v * 16, 16)] + rows_b[e, pl.ds(v * 16, 16)]
                rows_a[e, pl.ds(v * 16, 16)] = jnp.maximum(a, 0.2 * a)
            return 0

        lax.fori_loop(0, 128, _me, 0)
        pltpu.sync_copy(rows_a, me_o.at[pl.ds((base + r) * 128, 128)])
        return 0

    lax.fori_loop(0, RPW, _row, 0)
    plsc.subcore_barrier()

    for k in range(5):
        pltpu.sync_copy(mv_acc.at[pl.ds(s * 625 + k * 125, 125)],
                        mv_o.at[c, pl.ds(s * 625 + k * 125, 125)])
    pltpu.sync_copy(den_acc.at[pl.ds(s * 640, 640)],
                    den_o.at[c, pl.ds(s * 640, 640)])


_sc_edge = functools.partial(
    pl.kernel,
    out_type=[
        jax.ShapeDtypeStruct((EPAD, F), jnp.float32),
        jax.ShapeDtypeStruct((2, N, F), jnp.float32),
        jax.ShapeDtypeStruct((2, NPAD), jnp.float32),
    ],
    mesh=plsc.VectorSubcoreMesh(core_axis_name="c", subcore_axis_name="s",
                                num_cores=2, num_subcores=16),
    scratch_types=[
        pltpu.VMEM((N,), jnp.float32),
        pltpu.VMEM((RPW, 128), jnp.int32),
        pltpu.VMEM((RPW, 128), jnp.int32),
        pltpu.VMEM((RPW, 128), jnp.float32),
        pltpu.VMEM((128, F), jnp.float32),
        pltpu.VMEM((128, F), jnp.float32),
        pltpu.VMEM((128, F), jnp.float32),
        pltpu.VMEM((1, 128), jnp.float32),
        pltpu.VMEM((640,), jnp.float32),
        pltpu.VMEM_SHARED((N, F), jnp.float32),
        pltpu.VMEM_SHARED((NPAD,), jnp.float32),
        pltpu.SemaphoreType.DMA,
        pltpu.SemaphoreType.DMA,
        pltpu.SemaphoreType.DMA,
    ],
)


def _sc_edge_call(recv2, send2, hw_p, t1, lr, r_n, s_n):
    return _sc_edge(_sc_body)(recv2, send2, hw_p, t1, lr, r_n, s_n)


def _fin_body(mv2, den2, out):
    m = mv2[0] + mv2[1]
    d = den2[0] + den2[1]
    x = m / (d[:, None] + 1e-9)
    out[...] = jnp.where(x > 0, x, jnp.expm1(x))


def _tc_finalize(mv_part, den_part):
    nb = 1000
    return pl.pallas_call(
        _fin_body,
        grid=(N // nb,),
        in_specs=[
            pl.BlockSpec((2, nb, F), lambda i: (0, i, 0)),
            pl.BlockSpec((2, nb), lambda i: (0, i)),
        ],
        out_specs=pl.BlockSpec((nb, F), lambda i: (i, 0)),
        out_shape=jax.ShapeDtypeStruct((N, F), jnp.float32),
    )(mv_part, den_part)


def kernel(hv_ftr, he_ftr, p_ftr, q_ftr, edge_index,
           W_attend, b_attend, W_align, b_align, W_e, b_e):
    ei = edge_index.astype(jnp.int32)
    recv, send = ei[0], ei[1]

    wp, wq, whe = W_align[:F], W_align[F:2 * F], W_align[2 * F:]
    We1, We2, We3, We4 = (W_e[:F], W_e[F:2 * F], W_e[2 * F:3 * F], W_e[3 * F:])

    lr, r_n, s_n, t2, hw2 = _tc_precompute(
        hv_ftr, p_ftr, q_ftr, he_ftr, W_attend, b_attend,
        We1, We2, We3, We4, b_e, wp, wq, whe, b_align)

    pad = EPAD - E
    zi = jnp.zeros((pad,), jnp.int32)
    recv2 = jnp.concatenate([recv, zi]).reshape(ROWS_PAD, 128)
    send2 = jnp.concatenate([send, zi]).reshape(ROWS_PAD, 128)
    hw_p = jnp.concatenate([hw2[:, 0], jnp.full((pad,), -1e30, jnp.float32)]
                           ).reshape(ROWS_PAD, 128)

    me_pad, mv_part, den_part = _sc_edge_call(
        recv2, send2, hw_p, t2[:, 0], lr, r_n, s_n)

    mv_ftr = _tc_finalize(mv_part, den_part[:, :N])
    return mv_ftr, me_pad[:E]


# speedup vs baseline: 5.1606x; 5.1606x over previous
import functools

import jax
import jax.numpy as jnp
from jax import lax
from jax.experimental import pallas as pl
from jax.experimental.pallas import tpu as pltpu
from jax.experimental.pallas import tpu_sc as plsc

N = 10000
E = 320000
F = 128
NW = 32
RPW = 80
ROWS_PAD = NW * RPW
EPAD = ROWS_PAD * 128
NPAD = 10240


def _leaky2(x):
    return jnp.maximum(x, 0.2 * x)


def _pre_body(hv, p, q, he, Wa, ba, We1, We2, We3, We4, be, wp, wq, whe, bal,
              lr_o, r_o, s_o, t_o, hw_o):
    hvb, pb, qb = hv[...], p[...], q[...]
    p2 = jnp.dot(pb, We2[...])
    q3 = jnp.dot(qb, We3[...])
    lr_o[...] = _leaky2(jnp.dot(hvb, Wa[...]) + ba[...])
    r_o[...] = jnp.dot(hvb, We1[...]) - p2 - q3 + be[...]
    s_o[...] = p2 + q3 + jnp.dot(hvb, We4[...])
    t_o[...] = jnp.dot(pb, wp[...]) + jnp.dot(qb, wq[...])
    hw_o[...] = jnp.dot(he[...], whe[...]) + bal[...]


def _tc_precompute(hv, p, q, he, Wa, ba, We1, We2, We3, We4, be, wp, wq, whe, bal):
    nb = 1000
    eb = 4000
    grid = (N // nb,)
    node_in = pl.BlockSpec((nb, F), lambda i: (i, 0))
    full = lambda shape: pl.BlockSpec(shape, lambda i: tuple(0 for _ in shape))
    return pl.pallas_call(
        _pre_body,
        grid=grid,
        in_specs=[
            node_in, node_in, node_in,
            pl.BlockSpec((eb, 128), lambda i: (i, 0)),
            full((F, F)), full((F,)),
            full((F, F)), full((F, F)), full((F, F)), full((F, F)), full((F,)),
            full((F, 1)), full((F, 1)), full((128, 8)), full((1,)),
        ],
        out_specs=[
            pl.BlockSpec((nb, F), lambda i: (i, 0)),
            pl.BlockSpec((nb, F), lambda i: (i, 0)),
            pl.BlockSpec((nb, F), lambda i: (i, 0)),
            pl.BlockSpec((nb, 1), lambda i: (i, 0)),
            pl.BlockSpec((eb, 8), lambda i: (i, 0)),
        ],
        out_shape=[
            jax.ShapeDtypeStruct((N, F), jnp.float32),
            jax.ShapeDtypeStruct((N, F), jnp.float32),
            jax.ShapeDtypeStruct((N, F), jnp.float32),
            jax.ShapeDtypeStruct((N, 1), jnp.float32),
            jax.ShapeDtypeStruct((E // 128 * 16, 8), jnp.float32),
        ],
    )(hv, p, q, he, Wa, ba, We1, We2, We3, We4, be, wp, wq, whe, bal)


def _sc_body(recv2, send2, hw2, t_hbm, lr_hbm, r_hbm, s_hbm,
             me_o, mv_o, den_o,
             t_v, ridx8, sidx8, hw8, rows_a, rows_b, exrow, zden,
             mv_acc, den_acc, sem1, sem2):
    c = lax.axis_index("c")
    s = lax.axis_index("s")
    wid = c * 16 + s
    base = wid * RPW

    zv = jnp.zeros((16,), jnp.float32)

    def _zrow(e, _):
        for v in range(8):
            rows_a[e, pl.ds(v * 16, 16)] = zv
        return 0

    lax.fori_loop(0, 128, _zrow, 0)

    def _zden(i, _):
        zden[pl.ds(i * 16, 16)] = zv
        return 0

    lax.fori_loop(0, 40, _zden, 0)

    for k in range(5):
        pltpu.sync_copy(rows_a, mv_acc.at[pl.ds(s * 640 + k * 128, 128)])
    pltpu.sync_copy(zden, den_acc.at[pl.ds(s * 640, 640)])
    plsc.subcore_barrier()

    pltpu.sync_copy(t_hbm, t_v.at[pl.ds(0, N)])
    zi = jnp.zeros((16,), jnp.int32)

    def _group_a(g, _):
        row0 = base + g * 8
        pltpu.sync_copy(recv2.at[pl.ds(row0, 8)], ridx8)
        pltpu.sync_copy(send2.at[pl.ds(row0, 8)], sidx8)
        pltpu.sync_copy(hw2.at[pl.ds(row0, 8)], hw8)

        def _row_a(j, _):
            si_ref = sidx8.at[j]
            ri_ref = ridx8.at[j]
            c_l = pltpu.async_copy(lr_hbm.at[si_ref], rows_a, sem1)

            def _ex(k, _):
                si = sidx8[j, pl.ds(k * 16, 16)]
                ri = ridx8[j, pl.ds(k * 16, 16)]
                ts = plsc.load_gather(t_v, [si])
                tr = plsc.load_gather(t_v, [ri])
                hwv = hw8[j, pl.ds(k * 16, 16)]
                exrow[0, pl.ds(k * 16, 16)] = jnp.exp(ts - tr + hwv)
                return 0

            lax.fori_loop(0, 8, _ex, 0)
            pltpu.sync_copy(exrow.at[0], den_acc.at[ri_ref], add=True)
            c_l.wait()

            def _scale(e, _):
                exb = plsc.load_gather(exrow, [zi, jnp.full((16,), e, jnp.int32)])
                for v in range(8):
                    rows_a[e, pl.ds(v * 16, 16)] = rows_a[e, pl.ds(v * 16, 16)] * exb
                return 0

            lax.fori_loop(0, 128, _scale, 0)
            pltpu.sync_copy(rows_a, mv_acc.at[ri_ref], add=True)
            return 0

        lax.fori_loop(0, 8, _row_a, 0)
        return 0

    lax.fori_loop(0, RPW // 8, _group_a, 0)

    def _group_b(g, _):
        row0 = base + g * 8
        pltpu.sync_copy(recv2.at[pl.ds(row0, 8)], ridx8)
        pltpu.sync_copy(send2.at[pl.ds(row0, 8)], sidx8)

        def _row_b(j, _):
            c_a = pltpu.async_copy(r_hbm.at[ridx8.at[j]], rows_a, sem1)
            c_b = pltpu.async_copy(s_hbm.at[sidx8.at[j]], rows_b, sem2)
            c_a.wait()
            c_b.wait()

            def _me(e, _):
                for v in range(8):
                    a = rows_a[e, pl.ds(v * 16, 16)] + rows_b[e, pl.ds(v * 16, 16)]
                    rows_a[e, pl.ds(v * 16, 16)] = jnp.maximum(a, 0.2 * a)
                return 0

            lax.fori_loop(0, 128, _me, 0)
            pltpu.sync_copy(rows_a, me_o.at[pl.ds((row0 + j) * 128, 128)])
            return 0

        lax.fori_loop(0, 8, _row_b, 0)
        return 0

    lax.fori_loop(0, RPW // 8, _group_b, 0)
    plsc.subcore_barrier()

    for k in range(5):
        pltpu.sync_copy(mv_acc.at[pl.ds(s * 640 + k * 128, 128)],
                        mv_o.at[c, pl.ds(s * 640 + k * 128, 128)])
    pltpu.sync_copy(den_acc.at[pl.ds(s * 640, 640)],
                    den_o.at[c, pl.ds(s * 640, 640)])


_sc_edge = functools.partial(
    pl.kernel,
    out_type=[
        jax.ShapeDtypeStruct((EPAD, F), jnp.float32),
        jax.ShapeDtypeStruct((2, NPAD, F), jnp.float32),
        jax.ShapeDtypeStruct((2, NPAD), jnp.float32),
    ],
    mesh=plsc.VectorSubcoreMesh(core_axis_name="c", subcore_axis_name="s",
                                num_cores=2, num_subcores=16),
    compiler_params=pltpu.CompilerParams(needs_layout_passes=False),
    scratch_types=[
        pltpu.VMEM((NPAD,), jnp.float32),
        pltpu.VMEM((8, 128), jnp.int32),
        pltpu.VMEM((8, 128), jnp.int32),
        pltpu.VMEM((8, 128), jnp.float32),
        pltpu.VMEM((128, F), jnp.float32),
        pltpu.VMEM((128, F), jnp.float32),
        pltpu.VMEM((1, 128), jnp.float32),
        pltpu.VMEM((640,), jnp.float32),
        pltpu.VMEM_SHARED((NPAD, F), jnp.float32),
        pltpu.VMEM_SHARED((NPAD,), jnp.float32),
        pltpu.SemaphoreType.DMA,
        pltpu.SemaphoreType.DMA,
    ],
)


def _sc_edge_call(recv2, send2, hw_p, t1, lr, r_n, s_n):
    return _sc_edge(_sc_body)(recv2, send2, hw_p, t1, lr, r_n, s_n)


def _fin_body(mv2, den2, out):
    m = mv2[0] + mv2[1]
    d = den2[0, :, :1] + den2[1, :, :1]
    x = m / (d + 1e-9)
    out[...] = jnp.where(x > 0, x, jnp.exp(jnp.minimum(x, 0.0)) - 1.0)


def _tc_finalize(mv_part, den_part):
    nb = 1000
    return pl.pallas_call(
        _fin_body,
        grid=(N // nb,),
        in_specs=[
            pl.BlockSpec((2, nb, F), lambda i: (0, i, 0)),
            pl.BlockSpec((2, nb, 1), lambda i: (0, i, 0)),
        ],
        out_specs=pl.BlockSpec((nb, F), lambda i: (i, 0)),
        out_shape=jax.ShapeDtypeStruct((N, F), jnp.float32),
    )(mv_part, den_part)


def kernel(hv_ftr, he_ftr, p_ftr, q_ftr, edge_index,
           W_attend, b_attend, W_align, b_align, W_e, b_e):
    ei = edge_index.astype(jnp.int32)
    recv, send = ei[0], ei[1]

    wp, wq, whe = W_align[:F], W_align[F:2 * F], W_align[2 * F:]
    We1, We2, We3, We4 = (W_e[:F], W_e[F:2 * F], W_e[2 * F:3 * F], W_e[3 * F:])
    he2 = he_ftr.reshape(E // 8, 128)
    w16 = jnp.kron(jnp.eye(8, dtype=jnp.float32), whe)

    lr, r_n, s_n, t2, hw2 = _tc_precompute(
        hv_ftr, p_ftr, q_ftr, he2, W_attend, b_attend,
        We1, We2, We3, We4, b_e, wp, wq, w16, b_align)

    pad = EPAD - E
    zi = jnp.zeros((pad,), jnp.int32)
    recv2 = jnp.concatenate([recv, zi]).reshape(ROWS_PAD, 128)
    send2 = jnp.concatenate([send, zi]).reshape(ROWS_PAD, 128)
    hw_p = jnp.concatenate([hw2.reshape(E), jnp.full((pad,), -1e30, jnp.float32)]
                           ).reshape(ROWS_PAD, 128)

    me_pad, mv_part, den_part = _sc_edge_call(
        recv2, send2, hw_p, t2[:, 0], lr, r_n, s_n)

    mv_ftr = _tc_finalize(mv_part[:, :N], den_part[:, :N, None])
    return mv_ftr, me_pad[:E]
